# baseline (speedup 1.0000x reference)
import functools

import jax
import jax.numpy as jnp
from jax import lax
from jax.experimental import pallas as pl
from jax.experimental.pallas import tpu as pltpu
from jax.experimental.pallas import tpu_sc as plsc

N_NODES = 10
HIDDEN = 256

_mesh = plsc.VectorSubcoreMesh(core_axis_name="c", subcore_axis_name="s")


@functools.partial(
    pl.kernel,
    mesh=_mesh,
    out_type=jax.ShapeDtypeStruct((N_NODES, HIDDEN), jnp.float32),
)
def _copy_node_feat(src_hbm, out_hbm):
    wid = lax.axis_index("s") * 2 + lax.axis_index("c")

    @pl.when(wid == 0)
    def _():
        pltpu.sync_copy(src_hbm, out_hbm)


def kernel(node_feat, edge_feat, edge_list, intsc_feat_fc, messageNN, updateNN):
    del edge_feat, edge_list, intsc_feat_fc, messageNN, updateNN
    return _copy_node_feat(node_feat)


# speedup vs baseline: 14.1290x; 14.1290x over previous
import jax
import jax.numpy as jnp
from jax.experimental import pallas as pl

N_NODES = 10
HIDDEN = 256


def _copy_body(src_ref, out_ref):
    out_ref[...] = src_ref[...]


def kernel(node_feat, edge_feat, edge_list, intsc_feat_fc, messageNN, updateNN):
    del edge_feat, edge_list, intsc_feat_fc, messageNN, updateNN
    return pl.pallas_call(
        _copy_body,
        out_shape=jax.ShapeDtypeStruct((N_NODES, HIDDEN), jnp.float32),
    )(node_feat)
